# Initial kernel scaffold; baseline (speedup 1.0000x reference)
import jax
import jax.numpy as jnp
from jax.experimental import pallas as pl


def kernel(node_features, global_features, W1, b1, W2, b2, group_mask_nodes, batch, maneuver_mask):
    raise NotImplementedError("write your pallas kernel here")


# speedup vs baseline: 10.1663x; 10.1663x over previous
import functools

import jax
import jax.numpy as jnp
from jax import lax
from jax.experimental import pallas as pl
from jax.experimental.pallas import tpu as pltpu
from jax.experimental.pallas import tpu_sc as plsc

N = 32768
B = 16
MAX_G = 8
NODE_DIM = 512
HIDDEN = 1024
NUM_DIR = 7
NSLOT = B * MAX_G

NSUB = 16
CHUNK = N // NSUB
VECS = CHUNK // 16
GATHER_WORKERS = 8
NEG = -1000000000.0


def _sc_body(keep_hbm, batch_hbm, node_hbm,
             selp1_hbm, rows_hbm,
             batch_v, keep_v, hist_v, allh_v, cnt_v, h16_v, pe_v,
             contrib_v, allc_v, selp1_v, idx_v, rows_v,
             sh_hist, sh_contrib, sem):
    c = lax.axis_index("c")
    s = lax.axis_index("s")
    base = s * CHUNK
    zeros16 = jnp.zeros((16,), jnp.int32)
    lane = lax.iota(jnp.int32, 16)

    pltpu.sync_copy(batch_hbm.at[pl.ds(base, CHUNK)], batch_v)
    pltpu.sync_copy(keep_hbm.at[pl.ds(base, CHUNK)], keep_v)

    hist_v[...] = zeros16

    def h_step(i, carry):
        b16 = batch_v[pl.ds(i * 16, 16)]
        k16 = keep_v[pl.ds(i * 16, 16)]
        plsc.addupdate_scatter(hist_v, [b16], k16)
        return carry

    lax.fori_loop(0, VECS, h_step, 0)

    pltpu.sync_copy(hist_v, sh_hist.at[s])
    plsc.subcore_barrier()
    pltpu.sync_copy(sh_hist, allh_v)

    def off_step(w, acc):
        return acc + allh_v[w]

    cnt_v[...] = lax.fori_loop(0, s, off_step, zeros16)

    for j in range(NSLOT // 16):
        contrib_v[pl.ds(j * 16, 16)] = zeros16

    def p2_step(i, carry):
        b16 = batch_v[pl.ds(i * 16, 16)]
        k16 = keep_v[pl.ds(i * 16, 16)]
        h16_v[...] = zeros16
        plsc.addupdate_scatter(h16_v, [b16], k16)
        h16 = h16_v[...]
        pe_v[...] = plsc.cumsum(h16) - h16
        less_b = plsc.load_gather(pe_v, [b16])
        ck_excl = plsc.cumsum(k16) - k16
        basec = plsc.load_gather(cnt_v, [b16])
        g16 = basec + ck_excl - less_b
        valid = jnp.logical_and(k16 > 0, g16 < MAX_G)
        tgt = jnp.where(valid, b16 * MAX_G + g16, 0)
        rowid = base + i * 16 + lane + 1
        plsc.store_scatter(contrib_v, [tgt], rowid, mask=valid)
        cnt_v[...] = cnt_v[...] + h16
        return carry

    lax.fori_loop(0, VECS, p2_step, 0)

    pltpu.sync_copy(contrib_v, sh_contrib.at[s])
    plsc.subcore_barrier()
    pltpu.sync_copy(sh_contrib, allc_v)
    for j in range(NSLOT // 16):
        def sum_step(w, acc):
            return acc + allc_v[w, pl.ds(j * 16, 16)]
        selp1_v[pl.ds(j * 16, 16)] = lax.fori_loop(0, NSUB, sum_step, zeros16)

    @pl.when(jnp.logical_and(c == 0, s == 0))
    def _():
        pltpu.sync_copy(selp1_v, selp1_hbm)

    @pl.when(jnp.logical_and(c == 0, s < GATHER_WORKERS))
    def _():
        sel16 = selp1_v[pl.ds(s * 16, 16)]
        idx_v[...] = jnp.maximum(sel16 - 1, 0)
        pltpu.async_copy(node_hbm.at[idx_v], rows_v, sem).wait()
        pltpu.sync_copy(rows_v, rows_hbm.at[pl.ds(s * 16, 16)])


@functools.partial(jax.jit, static_argnames=("interpret",))
def _sc_call(keep_i, batch_i, node_features, interpret=False):
    mesh = plsc.VectorSubcoreMesh(core_axis_name="c", subcore_axis_name="s",
                                  num_cores=2, num_subcores=NSUB)
    fn = pl.kernel(
        _sc_body,
        out_type=(
            jax.ShapeDtypeStruct((NSLOT,), jnp.int32),
            jax.ShapeDtypeStruct((NSLOT, NODE_DIM), jnp.float32),
        ),
        mesh=mesh,
        scratch_types=[
            pltpu.VMEM((CHUNK,), jnp.int32),
            pltpu.VMEM((CHUNK,), jnp.int32),
            pltpu.VMEM((16,), jnp.int32),
            pltpu.VMEM((NSUB, 16), jnp.int32),
            pltpu.VMEM((16,), jnp.int32),
            pltpu.VMEM((16,), jnp.int32),
            pltpu.VMEM((16,), jnp.int32),
            pltpu.VMEM((NSLOT,), jnp.int32),
            pltpu.VMEM((NSUB, NSLOT), jnp.int32),
            pltpu.VMEM((NSLOT,), jnp.int32),
            pltpu.VMEM((16,), jnp.int32),
            pltpu.VMEM((16, NODE_DIM), jnp.float32),
            pltpu.VMEM_SHARED((NSUB, 16), jnp.int32),
            pltpu.VMEM_SHARED((NSUB, NSLOT), jnp.int32),
            pltpu.SemaphoreType.DMA,
        ],
        compiler_params=pltpu.CompilerParams(needs_layout_passes=False),
        interpret=interpret,
    )
    return fn(keep_i, batch_i, node_features)


def _tc_body(g_ref, glob_ref, w1_ref, b1_ref, w2_ref, b2_ref, sel_ref,
             mm_ref, out_ref):
    x = g_ref[...]
    node_part = jnp.dot(x, w1_ref[:NODE_DIM, :],
                        preferred_element_type=jnp.float32)
    glob_part = jnp.dot(glob_ref[...], w1_ref[NODE_DIM:, :],
                        preferred_element_type=jnp.float32)
    rows = lax.broadcasted_iota(jnp.int32, (NSLOT, B), 0) // MAX_G
    cols = lax.broadcasted_iota(jnp.int32, (NSLOT, B), 1)
    expand = (rows == cols).astype(jnp.float32)
    gexp = jnp.dot(expand, glob_part, preferred_element_type=jnp.float32)
    h = jnp.maximum(node_part + gexp + b1_ref[...], 0.0)
    out = jnp.dot(h, w2_ref[...], preferred_element_type=jnp.float32)
    out = out + b2_ref[...]
    valid = sel_ref[...] > 0
    keepmask = jnp.logical_and(mm_ref[...] > 0.0, valid)
    out_ref[...] = jnp.where(keepmask, out, NEG)


@functools.partial(jax.jit, static_argnames=("interpret",))
def _tc_call(gathered, global_features, W1, b1, W2, b2, selp1, mm,
             interpret=False):
    return pl.pallas_call(
        _tc_body,
        out_shape=jax.ShapeDtypeStruct((NSLOT, NUM_DIR), jnp.float32),
        interpret=interpret,
    )(gathered, global_features, W1, b1.reshape(1, HIDDEN), W2,
      b2.reshape(1, NUM_DIR), selp1.reshape(NSLOT, 1),
      mm.reshape(NSLOT, NUM_DIR).astype(jnp.float32))


def kernel(node_features, global_features, W1, b1, W2, b2,
           group_mask_nodes, batch, maneuver_mask):
    keep_i = group_mask_nodes.astype(jnp.int32)
    batch_i = batch.astype(jnp.int32)
    selp1, gathered = _sc_call(keep_i, batch_i, node_features)
    out = _tc_call(gathered, global_features, W1, b1, W2, b2, selp1,
                   maneuver_mask)
    return out.reshape(B, MAX_G * NUM_DIR)
